# baseline (speedup 1.0000x reference)
import functools

import jax
import jax.numpy as jnp
from jax import lax
from jax.experimental import pallas as pl
from jax.experimental.pallas import tpu as pltpu
from jax.experimental.pallas import tpu_sc as plsc

BM = 256


def _route_body(x_ref, gw_ref, gb_ref, eidx_ref, grank_ref, gs_ref, be_ref,
                acc_ref, *, E, nblocks, nblk_pad):
    i = pl.program_id(0)

    @pl.when(i == 0)
    def _():
        acc_ref[...] = jnp.zeros_like(acc_ref)

    x = x_ref[...]
    glog = jnp.dot(x, gw_ref[...], preferred_element_type=jnp.float32) + gb_ref[...]
    eidx = jnp.argmax(glog, axis=-1).astype(jnp.int32)
    e_iota = lax.broadcasted_iota(jnp.int32, (1, E), 1)
    onehot = (eidx[:, None] == e_iota).astype(jnp.float32)
    r = lax.broadcasted_iota(jnp.int32, (BM, BM), 0)
    c = lax.broadcasted_iota(jnp.int32, (BM, BM), 1)
    tri = (c < r).astype(jnp.float32)
    ranks_local = jnp.dot(tri, onehot, preferred_element_type=jnp.float32)
    counts_prev = acc_ref[...]
    grank = jnp.sum(onehot * (ranks_local + counts_prev), axis=-1)
    acc_ref[...] = counts_prev + jnp.sum(onehot, axis=0, keepdims=True)

    eidx_ref[...] = eidx.reshape(1, 1, BM)
    grank_ref[...] = grank.astype(jnp.int32).reshape(1, 1, BM)

    counts = acc_ref[...]
    pad_blocks = jnp.ceil(counts / BM)
    re_ = lax.broadcasted_iota(jnp.int32, (E, E), 0)
    ce_ = lax.broadcasted_iota(jnp.int32, (E, E), 1)
    tinc = (re_ <= ce_).astype(jnp.float32)
    ends = jnp.dot(pad_blocks, tinc, preferred_element_type=jnp.float32)
    starts_rows = (ends - pad_blocks) * BM
    gs_ref[...] = jnp.concatenate(
        [starts_rows, jnp.zeros_like(starts_rows)], axis=-1).astype(jnp.int32)
    j_iota = lax.broadcasted_iota(jnp.int32, (nblk_pad, 1), 0)
    be = jnp.sum((j_iota >= ends.astype(jnp.int32)).astype(jnp.int32),
                 axis=-1)
    be_ref[...] = jnp.clip(be, 0, E - 1).reshape(1, nblk_pad)


def _route(inputs, gate_W, gate_b2, *, N, D, E, nblk_pad):
    nblocks = N // BM
    return pl.pallas_call(
        functools.partial(_route_body, E=E, nblocks=nblocks, nblk_pad=nblk_pad),
        grid=(nblocks,),
        in_specs=[
            pl.BlockSpec((BM, D), lambda i: (i, 0)),
            pl.BlockSpec((D, E), lambda i: (0, 0)),
            pl.BlockSpec((1, E), lambda i: (0, 0)),
        ],
        out_specs=[
            pl.BlockSpec((1, 1, BM), lambda i: (i, 0, 0)),
            pl.BlockSpec((1, 1, BM), lambda i: (i, 0, 0)),
            pl.BlockSpec((1, 16), lambda i: (0, 0)),
            pl.BlockSpec((1, nblk_pad), lambda i: (0, 0)),
        ],
        out_shape=[
            jax.ShapeDtypeStruct((nblocks, 1, BM), jnp.int32),
            jax.ShapeDtypeStruct((nblocks, 1, BM), jnp.int32),
            jax.ShapeDtypeStruct((1, 16), jnp.int32),
            jax.ShapeDtypeStruct((1, nblk_pad), jnp.int32),
        ],
        scratch_shapes=[pltpu.VMEM((1, E), jnp.float32)],
    )(inputs, gate_W, gate_b2)


def _make_scatter(N, D, NPAD):
    info = plsc.get_sparse_core_info()
    NC, NS = info.num_cores, info.num_subcores
    NW = NC * NS
    n_per_w = N // NW
    CH = 128
    n_ch = n_per_w // CH

    @functools.partial(
        pl.kernel,
        out_type=[
            jax.ShapeDtypeStruct((NPAD, D), jnp.float32),
            jax.ShapeDtypeStruct((N,), jnp.int32),
        ],
        mesh=plsc.VectorSubcoreMesh(core_axis_name="c", subcore_axis_name="s"),
        compiler_params=pltpu.CompilerParams(needs_layout_passes=False),
        scratch_types=[
            pltpu.VMEM((16,), jnp.int32),
            pltpu.VMEM((CH,), jnp.int32),
            pltpu.VMEM((CH,), jnp.int32),
            pltpu.VMEM((CH,), jnp.int32),
            pltpu.VMEM((CH, D), jnp.float32),
            pltpu.SemaphoreType.DMA,
        ],
    )
    def scatter_x(x_hbm, eidx_hbm, grank_hbm, gs_hbm, xs_hbm, dest_hbm,
                  gs_v, eidx_v, grank_v, dest_v, rows_v, sem):
        wid = lax.axis_index("s") * NC + lax.axis_index("c")
        base = wid * n_per_w
        pltpu.sync_copy(gs_hbm, gs_v)
        for ci in range(n_ch):
            off = base + ci * CH
            pltpu.sync_copy(eidx_hbm.at[pl.ds(off, CH)], eidx_v)
            pltpu.sync_copy(grank_hbm.at[pl.ds(off, CH)], grank_v)
            for k in range(CH // 16):
                e16 = eidx_v[pl.ds(k * 16, 16)]
                g16 = grank_v[pl.ds(k * 16, 16)]
                s16 = plsc.load_gather(gs_v, [e16])
                dest_v[pl.ds(k * 16, 16)] = s16 + g16
            pltpu.sync_copy(x_hbm.at[pl.ds(off, CH)], rows_v)
            pltpu.async_copy(rows_v, xs_hbm.at[dest_v], sem).wait()
            pltpu.sync_copy(dest_v, dest_hbm.at[pl.ds(off, CH)])

    return scatter_x


def _make_gather(N, C, NPAD):
    info = plsc.get_sparse_core_info()
    NC, NS = info.num_cores, info.num_subcores
    NW = NC * NS
    n_per_w = N // NW
    CH = 64
    n_ch = n_per_w // CH

    @functools.partial(
        pl.kernel,
        out_type=jax.ShapeDtypeStruct((N, C), jnp.float32),
        mesh=plsc.VectorSubcoreMesh(core_axis_name="c", subcore_axis_name="s"),
        scratch_types=[
            pltpu.VMEM((CH,), jnp.int32),
            pltpu.VMEM((CH, C), jnp.float32),
            pltpu.SemaphoreType.DMA,
        ],
    )
    def gather_y(y_hbm, dest_hbm, out_hbm, dest_v, rows_v, sem):
        wid = lax.axis_index("s") * NC + lax.axis_index("c")
        base = wid * n_per_w
        for ci in range(n_ch):
            off = base + ci * CH
            pltpu.sync_copy(dest_hbm.at[pl.ds(off, CH)], dest_v)
            pltpu.async_copy(y_hbm.at[dest_v], rows_v, sem).wait()
            pltpu.sync_copy(rows_v, out_hbm.at[pl.ds(off, CH)])

    return gather_y


def _mm_body(be_ref, xs_ref, w_ref, b_ref, o_ref):
    y = jnp.dot(xs_ref[...], w_ref[0], preferred_element_type=jnp.float32)
    o_ref[...] = jax.nn.softmax(y + b_ref[0], axis=-1)


def _grouped_mm(be_arr, x_sorted, expert_W, expert_b, *, D, C, NPAD):
    nblk = NPAD // BM
    grid_spec = pltpu.PrefetchScalarGridSpec(
        num_scalar_prefetch=1,
        grid=(nblk,),
        in_specs=[
            pl.BlockSpec((BM, D), lambda j, be: (j, 0)),
            pl.BlockSpec((1, D, C), lambda j, be: (be[j], 0, 0)),
            pl.BlockSpec((1, 1, C), lambda j, be: (be[j], 0, 0)),
        ],
        out_specs=pl.BlockSpec((BM, C), lambda j, be: (j, 0)),
    )
    return pl.pallas_call(
        _mm_body,
        grid_spec=grid_spec,
        out_shape=jax.ShapeDtypeStruct((NPAD, C), jnp.float32),
    )(be_arr, x_sorted, expert_W, expert_b.reshape(expert_b.shape[0], 1, C))


def kernel(inputs, expert_W, expert_b, gate_W, gate_b):
    N, D = inputs.shape
    E, _, C = expert_W.shape
    NPAD = N + E * BM
    nblk_pad = 128

    eidx3, grank3, gs2, be2 = _route(
        inputs, gate_W, gate_b.reshape(1, E), N=N, D=D, E=E, nblk_pad=nblk_pad)
    eidx = eidx3.reshape(N)
    grank = grank3.reshape(N)
    gs = gs2.reshape(16)
    be_arr = be2.reshape(nblk_pad)

    x_sorted, dest = _make_scatter(N, D, NPAD)(inputs, eidx, grank, gs)
    y_sorted = _grouped_mm(be_arr, x_sorted, expert_W, expert_b,
                           D=D, C=C, NPAD=NPAD)
    out = _make_gather(N, C, NPAD)(y_sorted, dest)
    return out
